# baseline (speedup 1.0000x reference)
import functools

import jax
import jax.numpy as jnp
from jax import lax
from jax.experimental import pallas as pl
from jax.experimental.pallas import tpu as pltpu
from jax.experimental.pallas import tpu_sc as plsc

F = 26
FIELD_DIM = 100000
D = 32
B = 1024
VROWS = F * FIELD_DIM

NC = 2
NS = 16
NW = NC * NS
LANE_TILES = B // 128
NBLK = F * LANE_TILES
BLK_PER_W = -(-NBLK // NW)
RING = 16


def _sc_gather_t(table_t, idx3d):
    mesh = plsc.VectorSubcoreMesh(core_axis_name="c", subcore_axis_name="s")

    @functools.partial(
        pl.kernel,
        mesh=mesh,
        compiler_params=pltpu.CompilerParams(needs_layout_passes=False),
        out_type=jax.ShapeDtypeStruct((F, D, B), jnp.float32),
        scratch_types=[
            pltpu.VMEM((1, 144), jnp.int32),
            pltpu.VMEM_SHARED((NS, RING, D, 128), jnp.float32),
            pltpu.VMEM((4, D, 128), jnp.float32),
            pltpu.VMEM((D, 128), jnp.float32),
            pltpu.SemaphoreType.DMA,
            pltpu.SemaphoreType.DMA,
            pltpu.SemaphoreType.DMA,
        ],
    )
    def k(table_hbm, idx_hbm, out_hbm, idx_v, spm_ring, strips_v, blk_v,
          sem_i, sem_g, sem_x):
        sid = lax.axis_index("s")
        wid = sid * NC + lax.axis_index("c")
        my_ring = spm_ring.at[sid]
        rows_lo = lax.iota(jnp.int32, 16)
        rows_hi = rows_lo + 16

        def read_r(c):
            return idx_v[0, pl.ds(c, 16)][0]

        def fire_hbm(c, r):
            base = pl.multiple_of((r >> 7) << 7, 128)
            pltpu.async_copy(
                table_hbm.at[:, pl.ds(base, 128)],
                my_ring.at[lax.rem(c, RING)],
                sem_g,
            )

        def drain_hbm():
            pltpu.make_async_copy(
                table_hbm.at[:, pl.ds(0, 128)], my_ring.at[0], sem_g
            ).wait()

        def fire_xbar(c):
            pltpu.async_copy(
                my_ring.at[lax.rem(c, RING)],
                strips_v.at[lax.rem(c, 4)],
                sem_x,
            )

        def drain_xbar():
            pltpu.make_async_copy(
                my_ring.at[0], strips_v.at[0], sem_x
            ).wait()

        def extract(c, r):
            lane = jnp.broadcast_to(r & 127, (16,))
            col = jnp.broadcast_to(c, (16,))
            strip = strips_v.at[lax.rem(c, 4)]
            lo = plsc.load_gather(strip, [rows_lo, lane])
            hi = plsc.load_gather(strip, [rows_hi, lane])
            plsc.store_scatter(blk_v, [rows_lo, col], lo)
            plsc.store_scatter(blk_v, [rows_hi, col], hi)

        for i in range(BLK_PER_W):
            beta = wid + NW * i

            @pl.when(beta < NBLK)
            def _do_block():
                fld = beta // LANE_TILES
                lt = beta % LANE_TILES
                pltpu.sync_copy(idx_hbm.at[beta], idx_v.at[:, pl.ds(0, 128)])

                def prime(c, carry):
                    fire_hbm(c, read_r(c))
                    return carry

                lax.fori_loop(0, RING, prime, 0)

                def step(c, carry):
                    drain_hbm()

                    @pl.when(c >= 4)
                    def _ext_prev():
                        drain_xbar()
                        extract(c - 4, read_r(c - 4))

                    fire_xbar(c)

                    @pl.when(jnp.logical_and(c >= 4, c - 4 + RING < 128))
                    def _refire():
                        fire_hbm(c - 4 + RING, read_r(c - 4 + RING))

                    return carry

                lax.fori_loop(0, 128, step, 0)

                def tail(c, carry):
                    drain_xbar()
                    extract(c, read_r(c))
                    return carry

                lax.fori_loop(124, 128, tail, 0)

                pltpu.async_copy(
                    blk_v,
                    out_hbm.at[fld, :, pl.ds(lt * 128, 128)],
                    sem_i,
                ).wait()

    return k(table_t, idx3d)


def _tc_fwfm(e, field_cov, bias):
    def body(cov_ref, bias_ref, e_ref, out_ref):
        acc = jnp.zeros((D, B), jnp.float32)
        for k in range(F):
            ek = e_ref[k]
            for l in range(k + 1, F):
                s = (cov_ref[k, l] + cov_ref[l, k]) * 0.5
                acc = acc + s * (ek * e_ref[l])
        logits = jnp.sum(acc, axis=0, keepdims=True)
        out_ref[...] = jax.nn.sigmoid(logits + bias_ref[0])

    return pl.pallas_call(
        body,
        out_shape=jax.ShapeDtypeStruct((1, B), jnp.float32),
        in_specs=[
            pl.BlockSpec(memory_space=pltpu.SMEM),
            pl.BlockSpec(memory_space=pltpu.SMEM),
            pl.BlockSpec(memory_space=pltpu.VMEM),
        ],
    )(field_cov, bias, e)


def kernel(x, emb_table, field_cov, bias):
    offs = (jnp.arange(F, dtype=jnp.int32) * FIELD_DIM)[:, None]
    idx = (x.T + offs).reshape(NBLK, 1, 128)
    e = _sc_gather_t(emb_table.T, idx)
    out = _tc_fwfm(e, field_cov, bias)
    return out.reshape(B)


# speedup vs baseline: 1.5620x; 1.5620x over previous
import functools

import jax
import jax.numpy as jnp
from jax import lax
from jax.experimental import pallas as pl
from jax.experimental.pallas import tpu as pltpu
from jax.experimental.pallas import tpu_sc as plsc

F = 26
FIELD_DIM = 100000
D = 32
B = 1024
VROWS = F * FIELD_DIM

NC = 2
NS = 16
NW = NC * NS
LANE_TILES = B // 128
NBLK = F * LANE_TILES
BLK_PER_W = -(-NBLK // NW)
RING = 16


def _sc_gather_t(table_t, idx3d):
    mesh = plsc.VectorSubcoreMesh(core_axis_name="c", subcore_axis_name="s")

    @functools.partial(
        pl.kernel,
        mesh=mesh,
        compiler_params=pltpu.CompilerParams(needs_layout_passes=False),
        out_type=jax.ShapeDtypeStruct((F, D, B), jnp.float32),
        scratch_types=[
            pltpu.VMEM((1, 144), jnp.int32),
            pltpu.VMEM((RING, D, 128), jnp.float32),
            pltpu.VMEM((D, 128), jnp.float32),
            pltpu.SemaphoreType.DMA,
            pltpu.SemaphoreType.DMA,
        ],
    )
    def k(table_hbm, idx_hbm, out_hbm, idx_v, strips_v, blk_v, sem_i, sem_g):
        wid = lax.axis_index("s") * NC + lax.axis_index("c")
        rows_lo = lax.iota(jnp.int32, 16)
        rows_hi = rows_lo + 16

        def read_r(c):
            return idx_v[0, pl.ds(c, 16)][0]

        def fire(c, r):
            base = pl.multiple_of((r >> 7) << 7, 128)
            pltpu.async_copy(
                table_hbm.at[:, pl.ds(base, 128)],
                strips_v.at[lax.rem(c, RING)],
                sem_g,
            )

        def drain_one():
            pltpu.make_async_copy(
                table_hbm.at[:, pl.ds(0, 128)], strips_v.at[0], sem_g
            ).wait()

        def extract(c, r):
            lane = jnp.broadcast_to(r & 127, (16,))
            col = jnp.broadcast_to(c, (16,))
            strip = strips_v.at[lax.rem(c, RING)]
            lo = plsc.load_gather(strip, [rows_lo, lane])
            hi = plsc.load_gather(strip, [rows_hi, lane])
            plsc.store_scatter(blk_v, [rows_lo, col], lo)
            plsc.store_scatter(blk_v, [rows_hi, col], hi)

        for i in range(BLK_PER_W):
            beta = wid + NW * i

            @pl.when(beta < NBLK)
            def _do_block():
                fld = beta // LANE_TILES
                lt = beta % LANE_TILES
                pltpu.sync_copy(idx_hbm.at[beta], idx_v.at[:, pl.ds(0, 128)])

                def prime(c, carry):
                    fire(c, read_r(c))
                    return carry

                lax.fori_loop(0, RING, prime, 0)

                def step(c, carry):
                    drain_one()
                    extract(c, read_r(c))
                    fire(c + RING, read_r(c + RING))
                    return carry

                lax.fori_loop(0, 128 - RING, step, 0)

                def tail(c, carry):
                    drain_one()
                    extract(c, read_r(c))
                    return carry

                lax.fori_loop(128 - RING, 128, tail, 0)

                pltpu.async_copy(
                    blk_v,
                    out_hbm.at[fld, :, pl.ds(lt * 128, 128)],
                    sem_i,
                ).wait()

    return k(table_t, idx3d)


def _tc_fwfm(e, field_cov, bias):
    def body(cov_ref, bias_ref, e_ref, out_ref):
        acc = jnp.zeros((D, B), jnp.float32)
        for k in range(F):
            ek = e_ref[k]
            for l in range(k + 1, F):
                s = (cov_ref[k, l] + cov_ref[l, k]) * 0.5
                acc = acc + s * (ek * e_ref[l])
        logits = jnp.sum(acc, axis=0, keepdims=True)
        out_ref[...] = jax.nn.sigmoid(logits + bias_ref[0])

    return pl.pallas_call(
        body,
        out_shape=jax.ShapeDtypeStruct((1, B), jnp.float32),
        in_specs=[
            pl.BlockSpec(memory_space=pltpu.SMEM),
            pl.BlockSpec(memory_space=pltpu.SMEM),
            pl.BlockSpec(memory_space=pltpu.VMEM),
        ],
    )(field_cov, bias, e)


def kernel(x, emb_table, field_cov, bias):
    offs = (jnp.arange(F, dtype=jnp.int32) * FIELD_DIM)[:, None]
    idx = (x.T + offs).reshape(NBLK, 1, 128)
    e = _sc_gather_t(emb_table.T, idx)
    out = _tc_fwfm(e, field_cov, bias)
    return out.reshape(B)
